# Initial kernel scaffold; baseline (speedup 1.0000x reference)
import jax
import jax.numpy as jnp
from jax.experimental import pallas as pl


def kernel(root_probs, edge_probs):
    raise NotImplementedError("write your pallas kernel here")


# speedup vs baseline: 22.0716x; 22.0716x over previous
import random as _pyrandom

import numpy as np
import jax
import jax.numpy as jnp
from jax import lax
from jax.experimental import pallas as pl
from jax.experimental.pallas import tpu as pltpu
import jax.experimental.pallas.tpu_sc as plsc

NWORDS = 16
W_SIZE = 1 << 20
CHUNK = 16384
CAPS = 1 << 16
NBLK = CAPS // 16

_rng = _pyrandom.Random(0)
_W_TABLE = np.array([_rng.getrandbits(32) for _ in range(W_SIZE)],
                    dtype=np.uint64).astype(np.uint32).view(np.int32)


def _pack_weights(n):
    j = np.arange(n)
    w = np.arange(NWORDS)
    sel = j[:, None] // 32 == w[None, :]
    lo = np.where(sel & (j[:, None] % 32 < 16),
                  (1 << (j[:, None] % 32)).astype(np.float64), 0.0)
    hi = np.where(sel & (j[:, None] % 32 >= 16),
                  (1 << (j[:, None] % 32 - 16)).astype(np.float64), 0.0)
    return lo.astype(np.float32), hi.astype(np.float32)


def _decisions_body(epl0, epl1, rpl0, rpl1, g, wlo, whi, out):
    k = pl.program_id(0)
    g0 = g[0, 0:1, :]
    g1 = g[0, 1:2, :]
    p0 = jnp.where(k == 0, rpl0[...], epl0[...])
    p1 = jnp.where(k == 0, rpl1[...], epl1[...])
    a0 = p0 + g0
    a1 = p1 + g1
    m = jnp.maximum(a0, a1)
    e0 = jnp.exp(a0 - m)
    e1 = jnp.exp(a1 - m)
    s = e0 + e1
    hard = ((e0 / s) >= (e1 / s)).astype(jnp.float32)
    lo = lax.dot_general(hard, wlo[...], (((1,), (0,)), ((), ())),
                         precision=lax.Precision.HIGHEST)
    hi = lax.dot_general(hard, whi[...], (((1,), (0,)), ((), ())),
                         precision=lax.Precision.HIGHEST)
    out[0] = lo.astype(jnp.int32) | (hi.astype(jnp.int32) << 16)


def _compute_decisions(epl0, epl1, rpl0b, rpl1b, g3, interpret=False):
    n = epl0.shape[0]
    nk = n + 1
    wlo, whi = _pack_weights(n)
    return pl.pallas_call(
        _decisions_body,
        grid=(nk,),
        in_specs=[
            pl.BlockSpec((n, n), lambda k: (0, 0)),
            pl.BlockSpec((n, n), lambda k: (0, 0)),
            pl.BlockSpec((n, n), lambda k: (0, 0)),
            pl.BlockSpec((n, n), lambda k: (0, 0)),
            pl.BlockSpec((1, 2, n), lambda k: (k, 0, 0)),
            pl.BlockSpec((n, NWORDS), lambda k: (0, 0)),
            pl.BlockSpec((n, NWORDS), lambda k: (0, 0)),
        ],
        out_specs=pl.BlockSpec((1, n, NWORDS), lambda k: (k, 0, 0)),
        out_shape=jax.ShapeDtypeStruct((nk, n, NWORDS), jnp.int32),
        interpret=interpret,
    )(epl0, epl1, rpl0b, rpl1b, g3, jnp.asarray(wlo), jnp.asarray(whi))


def _make_sc_body(n):
    ngrp = n // 16

    def _sc_body(hardx, wtab, dag, slots, c2, c1, c0, ancrows, wchunk,
                 hrow, rootw, drow, zrow, sampled):
        cid = lax.axis_index("c")
        sid = lax.axis_index("s")

        i16 = lax.iota(jnp.int32, 16)
        zero16 = jnp.zeros((16,), jnp.int32)
        neg16 = jnp.full((16,), -1, jnp.int32)
        zf16 = jnp.zeros((16,), jnp.float32)

        def lane_of(vec, lane):
            return jnp.sum(jnp.where(i16 == lane, vec, 0))

        def sload(ref, idx):
            return plsc.load_gather(ref, [jnp.broadcast_to(idx, (16,))])[0]

        def sstore(ref, idx, val):
            plsc.store_scatter(ref, [jnp.broadcast_to(idx, (16,))],
                               jnp.broadcast_to(val, (16,)), mask=i16 == 0)

        def bump(ref, idx, delta):
            base = idx & jnp.int32(-16)
            v = ref[pl.ds(base, 16)]
            ref[pl.ds(base, 16)] = v + jnp.where(i16 == (idx & 15), delta, 0)

        @pl.when((cid == 0) & (sid == 0))
        def _():
            def mset(b, _):
                slots[pl.ds(b * 16, 16)] = neg16
                return 0
            lax.fori_loop(0, NBLK, mset, 0)

            def zc2(b, _):
                c2[pl.ds(b * 16, 16)] = zero16
                return 0
            lax.fori_loop(0, NBLK // 16, zc2, 0)

            def zc1(b, _):
                c1[pl.ds(b * 16, 16)] = zero16
                return 0
            lax.fori_loop(0, NBLK // 256, zc1, 0)
            c0[pl.ds(0, 16)] = zero16

            def zs(b, _):
                sampled[b] = 0
                return 0
            lax.fori_loop(0, n, zs, 0)

            def zr(b, _):
                zrow[pl.ds(b * 16, 16)] = zf16
                return 0
            lax.fori_loop(0, ngrp, zr, 0)

            def ainit(i, _):
                v = jnp.where(i16 == (i >> 5),
                              lax.shift_left(jnp.int32(1), i & 31), jnp.int32(0))
                ancrows[pl.ds(i * 16, 16)] = v
                return 0
            lax.fori_loop(0, n, ainit, 0)

            pltpu.sync_copy(hardx.at[0, 0], rootw)
            pltpu.sync_copy(wtab.at[pl.ds(0, CHUNK)], wchunk)
            rootv = rootw[pl.ds(0, 16)]

            def rfill(j, c):
                tail, length = c
                wv = lane_of(rootv, j >> 5)
                bit = lax.shift_right_logical(wv, j & 31) & 1

                def app(cc):
                    t, ln = cc
                    sstore(slots, t, j)
                    bump(c2, t >> 4, 1)
                    bump(c1, t >> 8, 1)
                    bump(c0, t >> 12, 1)
                    return t + 1, ln + 1

                return lax.cond(bit == 1, app, lambda cc: cc, c)

            tail, length = lax.fori_loop(
                0, n, rfill, (jnp.int32(0), jnp.int32(0)))

            def compact(old_tail):
                def cb(t, wr):
                    v = sload(slots, t)

                    def mv(w):
                        sstore(slots, w, v)
                        return w + 1
                    return lax.cond(v >= 0, mv, lambda w: w, wr)
                wr = lax.fori_loop(0, old_tail, cb, jnp.int32(0))

                def clear(w, _):
                    base = w * 16
                    v = slots[pl.ds(base, 16)]
                    slots[pl.ds(base, 16)] = jnp.where(base + i16 >= wr,
                                                       jnp.int32(-1), v)
                    return 0
                lax.fori_loop(wr >> 4, NBLK, clear, 0)

                def r2(w, _):
                    bidx = w * 16 + i16
                    c2[pl.ds(w * 16, 16)] = jnp.clip(wr - (bidx << 4), 0, 16)
                    return 0
                lax.fori_loop(0, NBLK // 16, r2, 0)

                def r1(w, _):
                    bidx = w * 16 + i16
                    c1[pl.ds(w * 16, 16)] = jnp.clip(wr - (bidx << 8), 0, 256)
                    return 0
                lax.fori_loop(0, NBLK // 256, r1, 0)
                c0[pl.ds(0, 16)] = jnp.clip(wr - (i16 << 12), 0, 4096)
                return wr

            def pop_body(st):
                length, tail, wp, cbase, pcount = st

                need = (wp + 64) > (cbase + CHUNK)
                new_cbase = jnp.where(need, wp & jnp.int32(-8), cbase)

                @pl.when(need)
                def _():
                    off = pl.multiple_of(new_cbase, 8)
                    pltpu.sync_copy(wtab.at[pl.ds(off, CHUNK)], wchunk)
                cbase = new_cbase

                fl = length.astype(jnp.float32)
                kb = (lax.bitcast_convert_type(fl, jnp.int32) >> 23) - 126
                sh = 32 - kb

                def draw(w):
                    off = w - cbase
                    vec = wchunk[pl.ds(off & jnp.int32(-16), 16)]
                    y = lane_of(vec, off & 15)
                    return lax.shift_right_logical(y, sh)

                r0_ = draw(wp)
                wp = wp + 1

                def redraw(c):
                    _, w = c
                    return draw(w), w + 1

                r, wp = lax.while_loop(lambda c: c[0] >= length, redraw,
                                       (r0_, wp))

                v0 = c0[pl.ds(0, 16)]
                cs0 = plsc.cumsum(v0)
                m0 = cs0 <= r
                n0 = jnp.sum(m0.astype(jnp.int32))
                r = r - jnp.sum(jnp.where(m0, v0, 0))
                c0[pl.ds(0, 16)] = v0 - jnp.where(i16 == n0, 1, 0)
                s0 = n0

                v1 = c1[pl.ds(s0 * 16, 16)]
                cs1 = plsc.cumsum(v1)
                m1 = cs1 <= r
                n1 = jnp.sum(m1.astype(jnp.int32))
                r = r - jnp.sum(jnp.where(m1, v1, 0))
                c1[pl.ds(s0 * 16, 16)] = v1 - jnp.where(i16 == n1, 1, 0)
                s1 = s0 * 16 + n1

                v2 = c2[pl.ds(s1 * 16, 16)]
                cs2 = plsc.cumsum(v2)
                m2 = cs2 <= r
                n2 = jnp.sum(m2.astype(jnp.int32))
                r = r - jnp.sum(jnp.where(m2, v2, 0))
                c2[pl.ds(s1 * 16, 16)] = v2 - jnp.where(i16 == n2, 1, 0)
                s2 = s1 * 16 + n2

                sv = slots[pl.ds(s2 * 16, 16)]
                alive = sv >= 0
                csa = plsc.cumsum(alive.astype(jnp.int32))
                lanem = alive & (csa == (r + 1))
                i = jnp.sum(jnp.where(lanem, sv, 0))
                slots[pl.ds(s2 * 16, 16)] = jnp.where(lanem, jnp.int32(-1), sv)
                length = length - 1

                def do_process(args):
                    tail, length, pcount = args
                    p = pcount + 1
                    sampled[i] = 1
                    tail = lax.cond(tail + n > CAPS, compact, lambda t: t, tail)
                    pltpu.sync_copy(hardx.at[p, i], hrow)
                    h = hrow[pl.ds(0, 16)]
                    a = ancrows[pl.ds(i * 16, 16)]
                    rm = rootw[pl.ds(0, 16)]
                    child = h & ~a & ~rm
                    biti = jnp.where(i16 == (i >> 5),
                                     lax.shift_left(jnp.int32(1), i & 31),
                                     jnp.int32(0))
                    anew = a | biti
                    tail0 = tail

                    def gbody(g, tl):
                        wv = lane_of(child, g >> 1)
                        shb = (g & 1) * 16
                        bits = lax.shift_right_logical(
                            jnp.broadcast_to(wv, (16,)), i16 + shb) & 1
                        drow[pl.ds(g * 16, 16)] = bits.astype(jnp.float32)
                        csb = plsc.cumsum(bits)
                        pos = tl + csb - bits
                        plsc.store_scatter(slots, [pos], i16 + g * 16,
                                           mask=bits != 0)
                        return tl + jnp.sum(bits)

                    tail = lax.fori_loop(0, ngrp, gbody, tail)
                    pltpu.sync_copy(drow, dag.at[i])

                    def cadd(ref, lg2):
                        def wb(w, _):
                            bidx = w * 16 + i16
                            lo = jnp.maximum(bidx << lg2, tail0)
                            hi = jnp.minimum((bidx + 1) << lg2, tail)
                            add = jnp.maximum(hi - lo, 0)
                            ref[pl.ds(w * 16, 16)] = ref[pl.ds(w * 16, 16)] + add
                            return 0
                        lax.fori_loop(tail0 >> (lg2 + 4),
                                      ((tail - 1) >> (lg2 + 4)) + 1, wb, 0)

                    @pl.when(tail > tail0)
                    def _():
                        cadd(c2, 4)
                        cadd(c1, 8)
                        cadd(c0, 12)

                    def abody(t, _):
                        c = sload(slots, t)
                        arow = ancrows[pl.ds(c * 16, 16)]
                        ancrows[pl.ds(c * 16, 16)] = arow | anew
                        return 0
                    lax.fori_loop(tail0, tail, abody, 0)
                    return tail, length + (tail - tail0), p

                si = sampled[i]
                tail, length, pcount = lax.cond(
                    si == 0, do_process, lambda a: a, (tail, length, pcount))
                return length, tail, wp, cbase, pcount

            st = (length, tail, jnp.int32(0), jnp.int32(0), jnp.int32(0))
            lax.while_loop(lambda s: s[0] > 0, pop_body, st)

            def zfill(rr, _):
                @pl.when(sampled[rr] == 0)
                def _():
                    pltpu.sync_copy(zrow, dag.at[rr])
                return 0
            lax.fori_loop(0, n, zfill, 0)

    return _sc_body


def _sc_run(hardx, wtab, interpret=False):
    n = hardx.shape[1]
    mesh = plsc.VectorSubcoreMesh(core_axis_name="c", subcore_axis_name="s",
                                  num_cores=2, num_subcores=16)
    f = pl.kernel(
        _make_sc_body(n),
        out_type=jax.ShapeDtypeStruct((n, n), jnp.float32),
        mesh=mesh,
        interpret=interpret,
        compiler_params=pltpu.CompilerParams(needs_layout_passes=False),
        scratch_types=[
            pltpu.VMEM((CAPS,), jnp.int32),
            pltpu.VMEM((NBLK,), jnp.int32),
            pltpu.VMEM((NBLK // 16,), jnp.int32),
            pltpu.VMEM((16,), jnp.int32),
            pltpu.VMEM((n * 16,), jnp.int32),
            pltpu.VMEM((CHUNK,), jnp.int32),
            pltpu.VMEM((16,), jnp.int32),
            pltpu.VMEM((16,), jnp.int32),
            pltpu.VMEM((n,), jnp.float32),
            pltpu.VMEM((n,), jnp.float32),
            pltpu.SMEM((n,), jnp.int32),
        ],
    )
    return f(hardx, wtab)


def _pipeline(root_probs, edge_probs, interpret=False):
    n = root_probs.shape[0]

    key = jax.random.key(42)
    subs = []
    for _ in range(n + 1):
        key, sub = jax.random.split(key)
        subs.append(sub)
    g3 = jax.vmap(lambda s: jax.random.gumbel(s, (2, n), jnp.float32))(
        jnp.stack(subs))

    pr = jax.nn.sigmoid(root_probs)
    rpl0b = jnp.broadcast_to(jnp.log(pr), (n, n))
    rpl1b = jnp.broadcast_to(jnp.log(1.0 - pr), (n, n))
    pe = jax.nn.sigmoid(edge_probs)
    epl0 = jnp.log(pe)
    epl1 = jnp.log(1.0 - pe)

    hardx = _compute_decisions(epl0, epl1, rpl0b, rpl1b, g3,
                               interpret=interpret)
    wtab = jnp.asarray(_W_TABLE)
    return _sc_run(hardx, wtab, interpret=interpret)


def kernel(root_probs, edge_probs):
    return _pipeline(root_probs, edge_probs)


# speedup vs baseline: 31.6986x; 1.4362x over previous
import random as _pyrandom

import numpy as np
import jax
import jax.numpy as jnp
from jax import lax
from jax.experimental import pallas as pl
from jax.experimental.pallas import tpu as pltpu
import jax.experimental.pallas.tpu_sc as plsc

NWORDS = 16
W_SIZE = 1 << 20
CHUNK = 16384
CAPS = 1 << 16
NBLK = CAPS // 16

_rng = _pyrandom.Random(0)
_W_TABLE = np.array([_rng.getrandbits(32) for _ in range(W_SIZE)],
                    dtype=np.uint64).astype(np.uint32).view(np.int32)


def _threefry2x32(k0, k1, x0, x1):
    rotl = lambda x, d: np.uint32((int(x) << d | int(x) >> (32 - d)) & 0xFFFFFFFF)
    add = lambda a, b: np.uint32((int(a) + int(b)) & 0xFFFFFFFF)
    R0 = (13, 15, 26, 6)
    R1 = (17, 29, 16, 24)
    ks = (np.uint32(k0), np.uint32(k1),
          np.uint32(k0) ^ np.uint32(k1) ^ np.uint32(0x1BD11BDA))
    x0 = add(x0, ks[0])
    x1 = add(x1, ks[1])
    for r in range(5):
        rots = R0 if r % 2 == 0 else R1
        for i in range(4):
            x0 = add(x0, x1)
            x1 = rotl(x1, rots[i])
            x1 = np.uint32(x1 ^ x0)
        x0 = add(x0, ks[(r + 1) % 3])
        x1 = add(add(x1, ks[(r + 2) % 3]), r + 1)
    return x0, x1


def _subkey_chain(nkeys):
    k = (np.uint32(0), np.uint32(42))
    subs = np.zeros((nkeys, 2), np.uint32)
    for t in range(nkeys):
        nxt = _threefry2x32(k[0], k[1], np.uint32(0), np.uint32(0))
        sub = _threefry2x32(k[0], k[1], np.uint32(0), np.uint32(1))
        subs[t] = sub
        k = nxt
    return subs


def _pack_weights(n):
    j = np.arange(n)
    w = np.arange(NWORDS)
    sel = j[:, None] // 32 == w[None, :]
    lo = np.where(sel & (j[:, None] % 32 < 16),
                  (1 << (j[:, None] % 32)).astype(np.float64), 0.0)
    hi = np.where(sel & (j[:, None] % 32 >= 16),
                  (1 << (j[:, None] % 32 - 16)).astype(np.float64), 0.0)
    return lo.astype(np.float32), hi.astype(np.float32)


def _decisions_body(epl0, epl1, rpl0, rpl1, g, wlo, whi, out):
    k = pl.program_id(0)
    g0 = g[0, 0:1, :]
    g1 = g[0, 1:2, :]
    p0 = jnp.where(k == 0, rpl0[...], epl0[...])
    p1 = jnp.where(k == 0, rpl1[...], epl1[...])
    a0 = p0 + g0
    a1 = p1 + g1
    m = jnp.maximum(a0, a1)
    e0 = jnp.exp(a0 - m)
    e1 = jnp.exp(a1 - m)
    s = e0 + e1
    hard = ((e0 / s) >= (e1 / s)).astype(jnp.float32)
    lo = lax.dot_general(hard, wlo[...], (((1,), (0,)), ((), ())),
                         precision=lax.Precision.HIGHEST)
    hi = lax.dot_general(hard, whi[...], (((1,), (0,)), ((), ())),
                         precision=lax.Precision.HIGHEST)
    out[0] = lo.astype(jnp.int32) | (hi.astype(jnp.int32) << 16)


def _compute_decisions(epl0, epl1, rpl0b, rpl1b, g3, interpret=False):
    n = epl0.shape[0]
    nk = n + 1
    wlo, whi = _pack_weights(n)
    return pl.pallas_call(
        _decisions_body,
        grid=(nk,),
        in_specs=[
            pl.BlockSpec((n, n), lambda k: (0, 0)),
            pl.BlockSpec((n, n), lambda k: (0, 0)),
            pl.BlockSpec((n, n), lambda k: (0, 0)),
            pl.BlockSpec((n, n), lambda k: (0, 0)),
            pl.BlockSpec((1, 2, n), lambda k: (k, 0, 0)),
            pl.BlockSpec((n, NWORDS), lambda k: (0, 0)),
            pl.BlockSpec((n, NWORDS), lambda k: (0, 0)),
        ],
        out_specs=pl.BlockSpec((1, n, NWORDS), lambda k: (k, 0, 0)),
        out_shape=jax.ShapeDtypeStruct((nk, n, NWORDS), jnp.int32),
        interpret=interpret,
    )(epl0, epl1, rpl0b, rpl1b, g3, jnp.asarray(wlo), jnp.asarray(whi))


def _make_sc_body(n):
    ngrp = n // 16

    def _sc_body(hardx, wtab, dag, slots, c2, c1, c0, ancrows, wchunk,
                 hrow, rootw, drow, zrow, sampled):
        cid = lax.axis_index("c")
        sid = lax.axis_index("s")

        i16 = lax.iota(jnp.int32, 16)
        zero16 = jnp.zeros((16,), jnp.int32)
        neg16 = jnp.full((16,), -1, jnp.int32)
        zf16 = jnp.zeros((16,), jnp.float32)

        def lane_of(vec, lane):
            return vec.at[jnp.broadcast_to(lane, (16,))].get(
                mode="promise_in_bounds")[0]

        def sload(ref, idx):
            return plsc.load_gather(ref, [jnp.broadcast_to(idx, (16,))])[0]

        def sstore(ref, idx, val):
            plsc.store_scatter(ref, [jnp.broadcast_to(idx, (16,))],
                               jnp.broadcast_to(val, (16,)), mask=i16 == 0)

        def bump(ref, idx, delta):
            base = idx & jnp.int32(-16)
            v = ref[pl.ds(base, 16)]
            ref[pl.ds(base, 16)] = v + jnp.where(i16 == (idx & 15), delta, 0)

        @pl.when((cid == 0) & (sid == 0))
        def _():
            def mset(b, _):
                slots[pl.ds(b * 16, 16)] = neg16
                return 0
            lax.fori_loop(0, NBLK, mset, 0)

            def zc2(b, _):
                c2[pl.ds(b * 16, 16)] = zero16
                return 0
            lax.fori_loop(0, NBLK // 16, zc2, 0)

            def zc1(b, _):
                c1[pl.ds(b * 16, 16)] = zero16
                return 0
            lax.fori_loop(0, NBLK // 256, zc1, 0)
            c0[pl.ds(0, 16)] = zero16

            def zs(b, _):
                sampled[b] = 0
                return 0
            lax.fori_loop(0, n, zs, 0)

            def zr(b, _):
                zrow[pl.ds(b * 16, 16)] = zf16
                return 0
            lax.fori_loop(0, ngrp, zr, 0)

            def ainit(i, _):
                v = jnp.where(i16 == (i >> 5),
                              lax.shift_left(jnp.int32(1), i & 31), jnp.int32(0))
                ancrows[pl.ds(i * 16, 16)] = v
                return 0
            lax.fori_loop(0, n, ainit, 0)

            pltpu.sync_copy(hardx.at[0, 0], rootw)
            pltpu.sync_copy(wtab.at[pl.ds(0, CHUNK)], wchunk)
            rootv = rootw[pl.ds(0, 16)]

            def rfill(j, c):
                tail, length = c
                wv = lane_of(rootv, j >> 5)
                bit = lax.shift_right_logical(wv, j & 31) & 1

                def app(cc):
                    t, ln = cc
                    sstore(slots, t, j)
                    bump(c2, t >> 4, 1)
                    bump(c1, t >> 8, 1)
                    bump(c0, t >> 12, 1)
                    return t + 1, ln + 1

                return lax.cond(bit == 1, app, lambda cc: cc, c)

            tail, length = lax.fori_loop(
                0, n, rfill, (jnp.int32(0), jnp.int32(0)))

            def compact(old_tail):
                def cb(t, wr):
                    v = sload(slots, t)

                    def mv(w):
                        sstore(slots, w, v)
                        return w + 1
                    return lax.cond(v >= 0, mv, lambda w: w, wr)
                wr = lax.fori_loop(0, old_tail, cb, jnp.int32(0))

                def clear(w, _):
                    base = w * 16
                    v = slots[pl.ds(base, 16)]
                    slots[pl.ds(base, 16)] = jnp.where(base + i16 >= wr,
                                                       jnp.int32(-1), v)
                    return 0
                lax.fori_loop(wr >> 4, NBLK, clear, 0)

                def r2(w, _):
                    bidx = w * 16 + i16
                    c2[pl.ds(w * 16, 16)] = jnp.clip(wr - (bidx << 4), 0, 16)
                    return 0
                lax.fori_loop(0, NBLK // 16, r2, 0)

                def r1(w, _):
                    bidx = w * 16 + i16
                    c1[pl.ds(w * 16, 16)] = jnp.clip(wr - (bidx << 8), 0, 256)
                    return 0
                lax.fori_loop(0, NBLK // 256, r1, 0)
                c0[pl.ds(0, 16)] = jnp.clip(wr - (i16 << 12), 0, 4096)
                return wr

            def pop_body(st):
                length, tail, wp, cbase, pcount = st

                need = (wp + 64) > (cbase + CHUNK)
                new_cbase = jnp.where(need, wp & jnp.int32(-8), cbase)

                @pl.when(need)
                def _():
                    off = pl.multiple_of(new_cbase, 8)
                    pltpu.sync_copy(wtab.at[pl.ds(off, CHUNK)], wchunk)
                cbase = new_cbase

                fl = length.astype(jnp.float32)
                kb = (lax.bitcast_convert_type(fl, jnp.int32) >> 23) - 126
                sh = 32 - kb

                def draw(w):
                    off = w - cbase
                    vec = wchunk[pl.ds(off & jnp.int32(-16), 16)]
                    y = lane_of(vec, off & 15)
                    return lax.shift_right_logical(y, sh)

                r0_ = draw(wp)
                wp = wp + 1

                def redraw(c):
                    _, w = c
                    return draw(w), w + 1

                r, wp = lax.while_loop(lambda c: c[0] >= length, redraw,
                                       (r0_, wp))

                def level(ref, base, r):
                    v = ref[pl.ds(base, 16)]
                    cs = plsc.cumsum(v)
                    nl = jnp.reshape(plsc.all_reduce_ffs(cs > r), (16,))[0]
                    r = r - lane_of(cs - v, nl)
                    ref[pl.ds(base, 16)] = v - jnp.where(i16 == nl, 1, 0)
                    return nl, r

                n0, r = level(c0, jnp.int32(0), r)
                s0 = n0
                n1, r = level(c1, s0 * 16, r)
                s1 = s0 * 16 + n1
                n2, r = level(c2, s1 * 16, r)
                s2 = s1 * 16 + n2

                sv = slots[pl.ds(s2 * 16, 16)]
                alive = sv >= 0
                csa = plsc.cumsum(alive.astype(jnp.int32))
                lanem = alive & (csa == (r + 1))
                lane = jnp.reshape(plsc.all_reduce_ffs(lanem), (16,))[0]
                i = lane_of(sv, lane)
                slots[pl.ds(s2 * 16, 16)] = jnp.where(i16 == lane,
                                                      jnp.int32(-1), sv)
                length = length - 1

                def do_process(args):
                    tail, length, pcount = args
                    p = pcount + 1
                    sampled[i] = 1
                    tail = lax.cond(tail + n > CAPS, compact, lambda t: t, tail)
                    pltpu.sync_copy(hardx.at[p, i], hrow)
                    h = hrow[pl.ds(0, 16)]
                    a = ancrows[pl.ds(i * 16, 16)]
                    rm = rootw[pl.ds(0, 16)]
                    child = h & ~a & ~rm
                    biti = jnp.where(i16 == (i >> 5),
                                     lax.shift_left(jnp.int32(1), i & 31),
                                     jnp.int32(0))
                    anew = a | biti
                    tail0 = tail

                    def gbody(g, tl):
                        wv = lane_of(child, g >> 1)
                        shb = (g & 1) * 16
                        bits = lax.shift_right_logical(
                            jnp.broadcast_to(wv, (16,)), i16 + shb) & 1
                        drow[pl.ds(g * 16, 16)] = bits.astype(jnp.float32)
                        csb = plsc.cumsum(bits)
                        pos = tl + csb - bits
                        plsc.store_scatter(slots, [pos], i16 + g * 16,
                                           mask=bits != 0)
                        return tl + csb[15]

                    tail = lax.fori_loop(0, ngrp, gbody, tail)
                    pltpu.sync_copy(drow, dag.at[i])

                    def cadd(ref, lg2):
                        def wb(w, _):
                            bidx = w * 16 + i16
                            lo = jnp.maximum(bidx << lg2, tail0)
                            hi = jnp.minimum((bidx + 1) << lg2, tail)
                            add = jnp.maximum(hi - lo, 0)
                            ref[pl.ds(w * 16, 16)] = ref[pl.ds(w * 16, 16)] + add
                            return 0
                        lax.fori_loop(tail0 >> (lg2 + 4),
                                      ((tail - 1) >> (lg2 + 4)) + 1, wb, 0)

                    @pl.when(tail > tail0)
                    def _():
                        cadd(c2, 4)
                        cadd(c1, 8)
                        cadd(c0, 12)

                    def abody(t, _):
                        c = sload(slots, t)
                        arow = ancrows[pl.ds(c * 16, 16)]
                        ancrows[pl.ds(c * 16, 16)] = arow | anew
                        return 0
                    lax.fori_loop(tail0, tail, abody, 0)
                    return tail, length + (tail - tail0), p

                si = sampled[i]
                tail, length, pcount = lax.cond(
                    si == 0, do_process, lambda a: a, (tail, length, pcount))
                return length, tail, wp, cbase, pcount

            st = (length, tail, jnp.int32(0), jnp.int32(0), jnp.int32(0))
            lax.while_loop(lambda s: s[0] > 0, pop_body, st)

            def zfill(rr, _):
                @pl.when(sampled[rr] == 0)
                def _():
                    pltpu.sync_copy(zrow, dag.at[rr])
                return 0
            lax.fori_loop(0, n, zfill, 0)

    return _sc_body


def _sc_run(hardx, wtab, interpret=False):
    n = hardx.shape[1]
    mesh = plsc.VectorSubcoreMesh(core_axis_name="c", subcore_axis_name="s",
                                  num_cores=2, num_subcores=16)
    f = pl.kernel(
        _make_sc_body(n),
        out_type=jax.ShapeDtypeStruct((n, n), jnp.float32),
        mesh=mesh,
        interpret=interpret,
        compiler_params=pltpu.CompilerParams(needs_layout_passes=False),
        scratch_types=[
            pltpu.VMEM((CAPS,), jnp.int32),
            pltpu.VMEM((NBLK,), jnp.int32),
            pltpu.VMEM((NBLK // 16,), jnp.int32),
            pltpu.VMEM((16,), jnp.int32),
            pltpu.VMEM((n * 16,), jnp.int32),
            pltpu.VMEM((CHUNK,), jnp.int32),
            pltpu.VMEM((16,), jnp.int32),
            pltpu.VMEM((16,), jnp.int32),
            pltpu.VMEM((n,), jnp.float32),
            pltpu.VMEM((n,), jnp.float32),
            pltpu.SMEM((n,), jnp.int32),
        ],
    )
    return f(hardx, wtab)


def _pipeline(root_probs, edge_probs, interpret=False):
    n = root_probs.shape[0]

    subs = jax.random.wrap_key_data(jnp.asarray(_subkey_chain(n + 1)),
                                    impl="threefry2x32")
    g3 = jax.vmap(lambda s: jax.random.gumbel(s, (2, n), jnp.float32))(subs)

    pr = jax.nn.sigmoid(root_probs)
    rpl0b = jnp.broadcast_to(jnp.log(pr), (n, n))
    rpl1b = jnp.broadcast_to(jnp.log(1.0 - pr), (n, n))
    pe = jax.nn.sigmoid(edge_probs)
    epl0 = jnp.log(pe)
    epl1 = jnp.log(1.0 - pe)

    hardx = _compute_decisions(epl0, epl1, rpl0b, rpl1b, g3,
                               interpret=interpret)
    wtab = jnp.asarray(_W_TABLE)
    return _sc_run(hardx, wtab, interpret=interpret)


def kernel(root_probs, edge_probs):
    return _pipeline(root_probs, edge_probs)
